# baseline (speedup 1.0000x reference)
import jax
import jax.numpy as jnp
from jax import lax
from jax.experimental import pallas as pl
from jax.experimental.pallas import tpu as pltpu
from jax.experimental.pallas import tpu_sc as plsc

B = 16384
D = 32
NC = 2
NS = 16
NW = NC * NS
BPW = B // NW
CHUNK = 128
NCHUNK = BPW // CHUNK


def _sc_gather_body(offu, offi, tgu, tgi, tmu, tmi,
                    out_gu, out_gi, out_mu, out_mi,
                    offu_v, offi_v, gu_v, gi_v, mu_v, mi_v,
                    sem0, sem1, sem2, sem3):
    wid = lax.axis_index("s") * NC + lax.axis_index("c")
    base = wid * BPW
    pltpu.sync_copy(offu.at[wid], offu_v)
    pltpu.sync_copy(offi.at[wid], offi_v)

    tabs = ((tgu, offu_v, gu_v, sem0), (tgi, offi_v, gi_v, sem1),
            (tmu, offu_v, mu_v, sem2), (tmi, offi_v, mi_v, sem3))

    def mk_fire(flat, off_v, dst_v, sem):
        def fire(f, _):
            for j in range(NCHUNK):
                pltpu.make_async_copy(
                    flat.at[off_v.at[f, j]],
                    dst_v.at[f, pl.ds(j * CHUNK, CHUNK)],
                    sem).start()
            return 0
        return fire

    def mk_drain(flat, dst_v, sem):
        def drain(f, _):
            pltpu.make_async_copy(flat.at[pl.ds(0, BPW)], dst_v.at[f], sem).wait()
            return 0
        return drain

    for flat, off_v, dst_v, sem in tabs:
        lax.fori_loop(0, D, mk_fire(flat, off_v, dst_v, sem), 0)
    for flat, off_v, dst_v, sem in tabs:
        lax.fori_loop(0, D, mk_drain(flat, dst_v, sem), 0)

    cols = pl.ds(base, BPW)
    pltpu.sync_copy(gu_v, out_gu.at[:, cols])
    pltpu.sync_copy(gi_v, out_gi.at[:, cols])
    pltpu.sync_copy(mu_v, out_mu.at[:, cols])
    pltpu.sync_copy(mi_v, out_mi.at[:, cols])


def _sc_gather(offu, offi, tgu, tgi, tmu, tmi):
    mesh = plsc.VectorSubcoreMesh(core_axis_name="c", subcore_axis_name="s",
                                  num_cores=NC, num_subcores=NS)
    f32 = jnp.float32
    out_type = tuple(jax.ShapeDtypeStruct((D, B), f32) for _ in range(4))
    scratch = [
        pltpu.VMEM((D, NCHUNK, CHUNK), jnp.int32),
        pltpu.VMEM((D, NCHUNK, CHUNK), jnp.int32),
        pltpu.VMEM((D, BPW), f32),
        pltpu.VMEM((D, BPW), f32),
        pltpu.VMEM((D, BPW), f32),
        pltpu.VMEM((D, BPW), f32),
        pltpu.SemaphoreType.DMA,
        pltpu.SemaphoreType.DMA,
        pltpu.SemaphoreType.DMA,
        pltpu.SemaphoreType.DMA,
    ]
    k = pl.kernel(_sc_gather_body, out_type=out_type, mesh=mesh,
                  scratch_types=scratch,
                  compiler_params=pltpu.CompilerParams(use_tc_tiling_on_sc=False))
    return k(offu, offi, tgu, tgi, tmu, tmi)


def _tc_head_body(gu, gi, mu, mi, wu, wi, w1a, w1b, b1, w2, b2, w3, b3,
                  w4, wfb, out):
    f32 = jnp.float32
    u = jnp.dot(wu[...], gu[...], preferred_element_type=f32) + wfb[0, 2]
    it = jnp.dot(wi[...], gi[...], preferred_element_type=f32) + wfb[0, 3]
    gmf = u * it
    x = jnp.dot(w1a[...], mu[...], preferred_element_type=f32)
    x = x + jnp.dot(w1b[...], mi[...], preferred_element_type=f32)
    x = jnp.maximum(x + b1[...], 0.0)
    x = jnp.maximum(jnp.dot(w2[...], x, preferred_element_type=f32) + b2[...], 0.0)
    x = jnp.maximum(jnp.dot(w3[...], x, preferred_element_type=f32) + b3[...], 0.0)
    mlp = jnp.dot(w4[...], x, preferred_element_type=f32)
    z = gmf * wfb[0, 0] + mlp * wfb[0, 1] + wfb[0, 4]
    out[...] = jax.nn.sigmoid(z)


def _tc_head(gu, gi, mu, mi, wu, wi, w1, b1, w2, b2, w3, b3, w4, b4,
             wf, bf, bu, bi):
    RB = 2048
    grid = (B // RB,)
    f32 = jnp.float32
    row_spec = pl.BlockSpec((D, RB), lambda i: (0, i))

    def rep(shape):
        return pl.BlockSpec(shape, lambda i: tuple(0 for _ in shape))

    w1aT = w1[:D].T
    w1bT = w1[D:].T
    wfb = jnp.stack([wf[0, 0], wf[1, 0], bu[0], bi[0],
                     b4[0] * wf[1, 0] + bf[0]]).reshape(1, 5)
    in_specs = [
        row_spec, row_spec, row_spec, row_spec,
        rep((1, D)), rep((1, D)),
        rep((16, D)), rep((16, D)), rep((16, 1)),
        rep((8, 16)), rep((8, 1)),
        rep((4, 8)), rep((4, 1)),
        rep((1, 4)), rep((1, 5)),
    ]
    out = pl.pallas_call(
        _tc_head_body,
        grid=grid,
        in_specs=in_specs,
        out_specs=pl.BlockSpec((1, RB), lambda i: (0, i)),
        out_shape=jax.ShapeDtypeStruct((1, B), f32),
    )(gu, gi, mu, mi, wu.reshape(1, D), wi.reshape(1, D),
      w1aT, w1bT, b1.reshape(16, 1), w2.T, b2.reshape(8, 1),
      w3.T, b3.reshape(4, 1), w4.reshape(1, 4), wfb)
    return out


def kernel(user_indices, item_indices, gmf_user_emb, gmf_item_emb, gmf_wu,
           gmf_bu, gmf_wi, gmf_bi, mlp_user_emb, mlp_item_emb, w1, b1, w2,
           b2, w3, b3, w4, b4, wf, bf):
    feat = (jnp.arange(D, dtype=jnp.int32) * 1000000).reshape(D, 1)
    uidx = user_indices.astype(jnp.int32).reshape(1, B)
    iidx = item_indices.astype(jnp.int32).reshape(1, B)
    offu = (feat + uidx).reshape(D, NW, NCHUNK * CHUNK).transpose(1, 0, 2) \
        .reshape(NW, D, NCHUNK, CHUNK)
    offi = (feat + iidx).reshape(D, NW, NCHUNK * CHUNK).transpose(1, 0, 2) \
        .reshape(NW, D, NCHUNK, CHUNK)
    gu, gi, mu, mi = _sc_gather(
        offu, offi,
        gmf_user_emb.T.reshape(-1), gmf_item_emb.T.reshape(-1),
        mlp_user_emb.T.reshape(-1), mlp_item_emb.T.reshape(-1))
    out = _tc_head(gu, gi, mu, mi, gmf_wu, gmf_wi, w1, b1, w2, b2, w3, b3,
                   w4, b4, wf, bf, gmf_bu, gmf_bi)
    return out.reshape(B)


# speedup vs baseline: 19.3665x; 19.3665x over previous
import jax
import jax.numpy as jnp
from jax import lax
from jax.experimental import pallas as pl
from jax.experimental.pallas import tpu as pltpu
from jax.experimental.pallas import tpu_sc as plsc

B = 16384
D = 32
V = 1000000
NC = 2
NS = 16
NW = NC * NS
BPW = B // NW
CHUNK = 64
NCHUNK = BPW // CHUNK
RW = 16
L = 16
PROWS = 7816
PITCH = PROWS * 128
KROWS = D * PITCH // RW
FPB = PITCH // RW


def _detile_body(src, out):
    out[...] = src[...].reshape(D, 512, 128)


def _detile(tT):
    grid = (16,)
    return pl.pallas_call(
        _detile_body,
        grid=grid,
        in_specs=[pl.BlockSpec((D, 65536), lambda c: (0, c))],
        out_specs=pl.BlockSpec((D, 512, 128), lambda c: (0, c, 0)),
        out_shape=jax.ShapeDtypeStruct((D, PROWS, 128), jnp.float32),
    )(tT)


def _sc_gather_body(krow_u, krow_i, m_u, m_i, tgu, tgi, tmu, tmi,
                    out_gu, out_gi, out_mu, out_mi,
                    ku_v, ki_v, mu_idx_v, mi_idx_v, buf, res_v, sem):
    wid = lax.axis_index("s") * NC + lax.axis_index("c")
    base = wid * BPW
    pltpu.sync_copy(krow_u.at[wid], ku_v)
    pltpu.sync_copy(krow_i.at[wid], ki_v)
    pltpu.sync_copy(m_u.at[wid], mu_idx_v)
    pltpu.sync_copy(m_i.at[wid], mi_idx_v)

    tabs = ((tgu, ku_v, mu_idx_v, out_gu), (tgi, ki_v, mi_idx_v, out_gi),
            (tmu, ku_v, mu_idx_v, out_mu), (tmi, ki_v, mi_idx_v, out_mi))

    for tab, k_v, m_v, out in tabs:
        for j in range(NCHUNK):
            def fire(f, _):
                pltpu.make_async_copy(
                    tab.at[k_v.at[f, j]], buf.at[f], sem).start()
                return 0
            lax.fori_loop(0, D, fire, 0)

            def drain(f, _):
                pltpu.make_async_copy(
                    tab.at[pl.ds(0, CHUNK)], buf.at[f], sem).wait()
                return 0
            lax.fori_loop(0, D, drain, 0)

            def extract(f, _):
                fv = jnp.full((L,), f, dtype=jnp.int32)
                for s in range(CHUNK // L):
                    rows = lax.iota(jnp.int32, L) + s * L
                    cols = m_v[j, pl.ds(s * L, L)]
                    vals = plsc.load_gather(buf, [fv, rows, cols])
                    res_v[f, pl.ds(j * CHUNK + s * L, L)] = vals
                return 0
            lax.fori_loop(0, D, extract, 0)
        pltpu.sync_copy(res_v, out.at[:, pl.ds(base, BPW)])


def _sc_gather(krow_u, krow_i, m_u, m_i, tgu, tgi, tmu, tmi):
    mesh = plsc.VectorSubcoreMesh(core_axis_name="c", subcore_axis_name="s",
                                  num_cores=NC, num_subcores=NS)
    f32 = jnp.float32
    i32 = jnp.int32
    out_type = tuple(jax.ShapeDtypeStruct((D, B), f32) for _ in range(4))
    scratch = [
        pltpu.VMEM((D, NCHUNK, CHUNK), i32),
        pltpu.VMEM((D, NCHUNK, CHUNK), i32),
        pltpu.VMEM((NCHUNK, CHUNK), i32),
        pltpu.VMEM((NCHUNK, CHUNK), i32),
        pltpu.VMEM((D, CHUNK, RW), f32),
        pltpu.VMEM((D, BPW), f32),
        pltpu.SemaphoreType.DMA,
    ]
    k = pl.kernel(_sc_gather_body, out_type=out_type, mesh=mesh,
                  scratch_types=scratch,
                  compiler_params=pltpu.CompilerParams(use_tc_tiling_on_sc=False,
                                                      needs_layout_passes=False))
    return k(krow_u, krow_i, m_u, m_i, tgu, tgi, tmu, tmi)


def _tc_head_body(gu, gi, mu, mi, wu, wi, w1a, w1b, b1, w2, b2, w3, b3,
                  w4, wfb, out):
    f32 = jnp.float32
    u = jnp.dot(wu[...], gu[...], preferred_element_type=f32) + wfb[0, 2]
    it = jnp.dot(wi[...], gi[...], preferred_element_type=f32) + wfb[0, 3]
    gmf = u * it
    x = jnp.dot(w1a[...], mu[...], preferred_element_type=f32)
    x = x + jnp.dot(w1b[...], mi[...], preferred_element_type=f32)
    x = jnp.maximum(x + b1[...], 0.0)
    x = jnp.maximum(jnp.dot(w2[...], x, preferred_element_type=f32) + b2[...], 0.0)
    x = jnp.maximum(jnp.dot(w3[...], x, preferred_element_type=f32) + b3[...], 0.0)
    mlp = jnp.dot(w4[...], x, preferred_element_type=f32)
    z = gmf * wfb[0, 0] + mlp * wfb[0, 1] + wfb[0, 4]
    out[...] = jax.nn.sigmoid(z)


def _tc_head(gu, gi, mu, mi, wu, wi, w1, b1, w2, b2, w3, b3, w4, b4,
             wf, bf, bu, bi):
    RB = 2048
    grid = (B // RB,)
    f32 = jnp.float32
    row_spec = pl.BlockSpec((D, RB), lambda i: (0, i))

    def rep(shape):
        return pl.BlockSpec(shape, lambda i: tuple(0 for _ in shape))

    w1aT = w1[:D].T
    w1bT = w1[D:].T
    wfb = jnp.stack([wf[0, 0], wf[1, 0], bu[0], bi[0],
                     b4[0] * wf[1, 0] + bf[0]]).reshape(1, 5)
    in_specs = [
        row_spec, row_spec, row_spec, row_spec,
        rep((1, D)), rep((1, D)),
        rep((16, D)), rep((16, D)), rep((16, 1)),
        rep((8, 16)), rep((8, 1)),
        rep((4, 8)), rep((4, 1)),
        rep((1, 4)), rep((1, 5)),
    ]
    out = pl.pallas_call(
        _tc_head_body,
        grid=grid,
        in_specs=in_specs,
        out_specs=pl.BlockSpec((1, RB), lambda i: (0, i)),
        out_shape=jax.ShapeDtypeStruct((1, B), f32),
    )(gu, gi, mu, mi, wu.reshape(1, D), wi.reshape(1, D),
      w1aT, w1bT, b1.reshape(16, 1), w2.T, b2.reshape(8, 1),
      w3.T, b3.reshape(4, 1), w4.reshape(1, 4), wfb)
    return out


def kernel(user_indices, item_indices, gmf_user_emb, gmf_item_emb, gmf_wu,
           gmf_bu, gmf_wi, gmf_bi, mlp_user_emb, mlp_item_emb, w1, b1, w2,
           b2, w3, b3, w4, b4, wf, bf):
    feat = (jnp.arange(D, dtype=jnp.int32) * FPB).reshape(D, 1)
    uidx = user_indices.astype(jnp.int32).reshape(1, B)
    iidx = item_indices.astype(jnp.int32).reshape(1, B)
    krow_u = (feat + uidx // RW).reshape(D, NW, BPW).transpose(1, 0, 2) \
        .reshape(NW, D, NCHUNK, CHUNK)
    krow_i = (feat + iidx // RW).reshape(D, NW, BPW).transpose(1, 0, 2) \
        .reshape(NW, D, NCHUNK, CHUNK)
    m_u = (user_indices.astype(jnp.int32) % RW).reshape(NW, NCHUNK, CHUNK)
    m_i = (item_indices.astype(jnp.int32) % RW).reshape(NW, NCHUNK, CHUNK)
    tabs = [_detile(t.T).reshape(KROWS, RW)
            for t in (gmf_user_emb, gmf_item_emb, mlp_user_emb, mlp_item_emb)]
    gu, gi, mu, mi = _sc_gather(krow_u, krow_i, m_u, m_i, *tabs)
    out = _tc_head(gu, gi, mu, mi, gmf_wu, gmf_wi, w1, b1, w2, b2, w3, b3,
                   w4, b4, wf, bf, gmf_bu, gmf_bi)
    return out.reshape(B)
